# baseline (speedup 1.0000x reference)
import jax
import jax.numpy as jnp
from jax import lax
from jax.experimental import pallas as pl
from jax.experimental.pallas import tpu as pltpu
from jax.experimental.pallas import tpu_sc as plsc

_MOM = 0.5
_EPS = 1e-07

_B = 1024
_F = 128
_KP1 = 1025
_N = 1000000

_NC = 2
_NS = 16
_NW = _NC * _NS
_BPW = _B // _NW
_CH = 128
_K = _KP1 - 1
_NCH = _K // _CH
_GB = 8
_BB = _B // _GB
_RPW = (_N // _NW) // 8 * 8
_RTAIL = _N - _RPW * _NW


def _norm_body(l_ref, ab_ref, ln_ref, abn_ref):
    l = l_ref[...]
    ab = ab_ref[...]
    ln_ref[...] = l / (jnp.sqrt(jnp.sum(l * l, axis=1, keepdims=True)) + _EPS)
    abn_ref[...] = ab / (jnp.sqrt(jnp.sum(ab * ab, axis=1, keepdims=True)) + _EPS)


def _sc_body(ln_hbm, abn_hbm, y_hbm, idxn_hbm, meml_hbm, memab_hbm,
             dpab_hbm, dpl_hbm, ppab_hbm, ppl_hbm, vl_hbm, vab_hbm,
             xn_l, xn_ab, y_v, idx_v, vl_v, vab_v, dsq_v, pos_v,
             buf0, buf1, sem0, sem1, vsem):
    cid = lax.axis_index("c")
    sid = lax.axis_index("s")
    wid = sid * _NC + cid
    base_b = wid * _BPW

    pltpu.sync_copy(ln_hbm.at[pl.ds(base_b, _BPW)], xn_l)
    pltpu.sync_copy(abn_hbm.at[pl.ds(base_b, _BPW)], xn_ab)
    pltpu.sync_copy(y_hbm.at[pl.ds(base_b, _BPW)], y_v)
    pltpu.sync_copy(idxn_hbm.at[pl.ds(wid * _BPW * _NCH, _BPW * _NCH)], idx_v)

    pltpu.async_copy(memab_hbm.at[y_v], vab_v, vsem).wait()
    pltpu.async_copy(meml_hbm.at[y_v], vl_v, vsem).wait()

    lanes = lax.broadcasted_iota(jnp.int32, (16,), 0)
    zero16 = jnp.zeros((16,), jnp.float32)

    def run_phase(mem_hbm, xn, v_v, dsq_hbm, pos_hbm):
        def pos_grp(g, carry):
            def pos_row(rr, tot):
                r = g * 16 + rr
                for j in range(_F // 16):
                    d = v_v[r, pl.ds(j * 16, 16)] - xn[r, pl.ds(j * 16, 16)]
                    acc = d * d if j == 0 else acc + d * d
                return jnp.where(lanes == rr, jnp.sum(acc), tot)
            tot = lax.fori_loop(0, 16, pos_row, zero16)
            pos_v[pl.ds(g * 16, 16)] = tot
            return carry
        lax.fori_loop(0, _BPW // 16, pos_grp, 0)
        pltpu.sync_copy(pos_v, pos_hbm.at[pl.ds(base_b, _BPW)])

        def nb_body(b, carry):
            xr = [xn[b, pl.ds(j * 16, 16)] for j in range(_F // 16)]
            pending_in = pltpu.async_copy(mem_hbm.at[idx_v.at[b * _NCH]],
                                          buf0, sem0)
            for c in range(_NCH):
                cur = pending_in
                bufc = buf0 if c % 2 == 0 else buf1
                if c + 1 < _NCH:
                    nbuf = buf1 if c % 2 == 0 else buf0
                    nsem = sem1 if c % 2 == 0 else sem0
                    pending_in = pltpu.async_copy(
                        mem_hbm.at[idx_v.at[b * _NCH + c + 1]], nbuf, nsem)
                cur.wait()

                def grp_body(g, gc, _bufc=bufc, _c=c):
                    def row_body(rr, tot):
                        r = g * 16 + rr
                        for j in range(_F // 16):
                            d = _bufc[r, pl.ds(j * 16, 16)] - xr[j]
                            acc = d * d if j == 0 else acc + d * d
                        return jnp.where(lanes == rr, jnp.sum(acc), tot)
                    tot = lax.fori_loop(0, 16, row_body, zero16)
                    dsq_v[pl.ds(b * _K + _c * _CH + g * 16, 16)] = tot
                    return gc
                lax.fori_loop(0, _CH // 16, grp_body, 0)
            return carry
        lax.fori_loop(0, _BPW, nb_body, 0)
        pltpu.sync_copy(dsq_v, dsq_hbm.at[pl.ds(base_b * _K, _BPW * _K)])

    run_phase(memab_hbm, xn_l, vab_v, dpab_hbm, ppab_hbm)
    run_phase(meml_hbm, xn_ab, vl_v, dpl_hbm, ppl_hbm)

    pltpu.sync_copy(vl_v, vl_hbm.at[pl.ds(base_b, _BPW)])
    pltpu.sync_copy(vab_v, vab_hbm.at[pl.ds(base_b, _BPW)])


def _sc_gather(l_n, ab_n, y, idx_neg, memory_l, memory_ab):
    mesh = plsc.VectorSubcoreMesh(core_axis_name="c", subcore_axis_name="s")
    f32 = jnp.float32
    out_type = [
        jax.ShapeDtypeStruct((_B * _K,), f32),
        jax.ShapeDtypeStruct((_B * _K,), f32),
        jax.ShapeDtypeStruct((_B,), f32),
        jax.ShapeDtypeStruct((_B,), f32),
        jax.ShapeDtypeStruct((_B, _F), f32),
        jax.ShapeDtypeStruct((_B, _F), f32),
    ]
    scratch = [
        pltpu.VMEM((_BPW, _F), f32),
        pltpu.VMEM((_BPW, _F), f32),
        pltpu.VMEM((_BPW,), jnp.int32),
        pltpu.VMEM((_BPW * _NCH, _CH), jnp.int32),
        pltpu.VMEM((_BPW, _F), f32),
        pltpu.VMEM((_BPW, _F), f32),
        pltpu.VMEM((_BPW * _K,), f32),
        pltpu.VMEM((_BPW,), f32),
        pltpu.VMEM((_CH, _F), f32),
        pltpu.VMEM((_CH, _F), f32),
        pltpu.SemaphoreType.DMA,
        pltpu.SemaphoreType.DMA,
        pltpu.SemaphoreType.DMA,
    ]
    fn = pl.kernel(_sc_body, out_type=out_type, mesh=mesh,
                   scratch_types=scratch,
                   compiler_params=pltpu.CompilerParams(
                       needs_layout_passes=False,
                       skip_device_barrier=True),
                   cost_estimate=pl.CostEstimate(
                       flops=600_000_000,
                       transcendentals=0,
                       bytes_accessed=1_100_000_000))
    return fn(l_n, ab_n, y, idx_neg, memory_l, memory_ab)


def _finish_body(dsqab_ref, dsql_ref, posab_ref, posl_ref,
                 outl_ref, outab_ref):
    one = jnp.ones((_BB, 1), jnp.float32)

    def finish(dsq_ref, pos_ref, out_ref):
        pos = jnp.sqrt(pos_ref[...]).reshape(_BB, 1)
        d = jnp.sqrt(dsq_ref[...])
        o = jnp.maximum(1.0 + pos - d, 0.0)
        out_ref[...] = jnp.concatenate([one, o], axis=1)

    finish(dsqab_ref, posab_ref, outl_ref)
    finish(dsql_ref, posl_ref, outab_ref)


def _tc_finish(dsqab, dsql, posab, posl):
    f32 = jnp.float32
    dspec = pl.BlockSpec((_BB, _K), lambda g: (g, 0))
    pspec = pl.BlockSpec((_BB,), lambda g: (g,))
    ospec = pl.BlockSpec((_BB, _KP1), lambda g: (g, 0))
    return pl.pallas_call(
        _finish_body,
        grid=(_GB,),
        in_specs=[dspec, dspec, pspec, pspec],
        out_specs=[ospec, ospec],
        out_shape=[
            jax.ShapeDtypeStruct((_B, _KP1), f32),
            jax.ShapeDtypeStruct((_B, _KP1), f32),
        ],
    )(dsqab, dsql, posab, posl)


_NCP = 8


def _scatter_body(ln_ref, abn_ref, y_ref, meml_ref, memab_ref,
                  nml_ref, nmab_ref,
                  vl_scr, vab_scr, updl_scr, updab_scr, csem, sem):
    rows_per_chunk = _N // _NCP
    copies = []
    for t in range(_NCP):
        sl = pl.ds(t * rows_per_chunk, rows_per_chunk)
        copies.append(pltpu.make_async_copy(meml_ref.at[sl], nml_ref.at[sl],
                                            csem))
        copies[-1].start()
        copies.append(pltpu.make_async_copy(memab_ref.at[sl], nmab_ref.at[sl],
                                            csem))
        copies[-1].start()

    def gissue(i, carry):
        yi = y_ref[i]
        pltpu.make_async_copy(meml_ref.at[yi], vl_scr.at[i], sem).start()
        pltpu.make_async_copy(memab_ref.at[yi], vab_scr.at[i], sem).start()
        return carry
    lax.fori_loop(0, _B, gissue, 0)

    def gdrain(i, carry):
        yi = y_ref[i]
        pltpu.make_async_copy(meml_ref.at[yi], vl_scr.at[i], sem).wait()
        pltpu.make_async_copy(memab_ref.at[yi], vab_scr.at[i], sem).wait()
        return carry
    lax.fori_loop(0, _B, gdrain, 0)

    wl = vl_scr[...] * _MOM + ln_ref[...] * (1.0 - _MOM)
    updl_scr[...] = wl / jnp.sqrt(jnp.sum(wl * wl, axis=1, keepdims=True))
    wab = vab_scr[...] * _MOM + abn_ref[...] * (1.0 - _MOM)
    updab_scr[...] = wab / jnp.sqrt(jnp.sum(wab * wab, axis=1, keepdims=True))

    for cp in copies:
        cp.wait()

    def issue(i, carry):
        yi = y_ref[i]
        pltpu.make_async_copy(updl_scr.at[i], nml_ref.at[yi], sem).start()
        pltpu.make_async_copy(updab_scr.at[i], nmab_ref.at[yi], sem).start()
        return carry
    lax.fori_loop(0, _B, issue, 0)

    def drain(i, carry):
        yi = y_ref[i]
        pltpu.make_async_copy(updl_scr.at[i], nml_ref.at[yi], sem).wait()
        pltpu.make_async_copy(updab_scr.at[i], nmab_ref.at[yi], sem).wait()
        return carry
    lax.fori_loop(0, _B, drain, 0)


def _tc_scatter(l_n, ab_n, y, memory_l, memory_ab):
    f32 = jnp.float32
    vspec = pl.BlockSpec(memory_space=pltpu.VMEM)
    aspec = pl.BlockSpec(memory_space=pl.ANY)
    sspec = pl.BlockSpec(memory_space=pltpu.SMEM)
    return pl.pallas_call(
        _scatter_body,
        in_specs=[vspec, vspec, sspec, aspec, aspec],
        out_specs=[aspec, aspec],
        out_shape=[
            jax.ShapeDtypeStruct((_N, _F), f32),
            jax.ShapeDtypeStruct((_N, _F), f32),
        ],
        scratch_shapes=[
            pltpu.VMEM((_B, _F), f32),
            pltpu.VMEM((_B, _F), f32),
            pltpu.VMEM((_B, _F), f32),
            pltpu.VMEM((_B, _F), f32),
            pltpu.SemaphoreType.DMA,
            pltpu.SemaphoreType.DMA,
        ],
    )(l_n, ab_n, y, memory_l, memory_ab)


def kernel(l, ab, y, idx, memory_l, memory_ab):
    l_n, ab_n = pl.pallas_call(
        _norm_body,
        out_shape=[jax.ShapeDtypeStruct((_B, _F), jnp.float32)] * 2,
    )(l, ab)

    idx_neg = idx[:, 1:].reshape(_B * _NCH, _CH)

    dsqab, dsql, posab, posl, _vl, _vab = _sc_gather(
        l_n, ab_n, y, idx_neg, memory_l, memory_ab)

    out_l, out_ab = _tc_finish(
        dsqab.reshape(_B, _K), dsql.reshape(_B, _K), posab, posl)

    nml, nmab = _tc_scatter(l_n, ab_n, y, memory_l, memory_ab)

    return (out_l[..., None], out_ab[..., None], nml, nmab)


# speedup vs baseline: 16.7798x; 16.7798x over previous
import jax
import jax.numpy as jnp
from jax import lax
from jax.experimental import pallas as pl
from jax.experimental.pallas import tpu as pltpu
from jax.experimental.pallas import tpu_sc as plsc

_MOM = 0.5
_EPS = 1e-07

_B = 1024
_F = 128
_KP1 = 1025
_N = 1000000

_NC = 2
_NS = 16
_NW = _NC * _NS
_BPW = _B // _NW
_CH = 128
_K = _KP1 - 1
_NCH = _K // _CH
_GB = 8
_BB = _B // _GB
_RPW = (_N // _NW) // 8 * 8
_RTAIL = _N - _RPW * _NW


def _norm_body(l_ref, ab_ref, ln_ref, abn_ref):
    l = l_ref[...]
    ab = ab_ref[...]
    ln_ref[...] = l / (jnp.sqrt(jnp.sum(l * l, axis=1, keepdims=True)) + _EPS)
    abn_ref[...] = ab / (jnp.sqrt(jnp.sum(ab * ab, axis=1, keepdims=True)) + _EPS)


def _sc_body(ln_hbm, abn_hbm, y_hbm, idxn_hbm, meml_hbm, memab_hbm,
             dpab_hbm, dpl_hbm, ppab_hbm, ppl_hbm, vl_hbm, vab_hbm,
             xn_l, xn_ab, y_v, idx_v, vl_v, vab_v, dsq_v, pos_v, part,
             buf0, buf1, sem0, sem1, vsem):
    cid = lax.axis_index("c")
    sid = lax.axis_index("s")
    wid = sid * _NC + cid
    base_b = wid * _BPW

    pltpu.sync_copy(ln_hbm.at[pl.ds(base_b, _BPW)], xn_l)
    pltpu.sync_copy(abn_hbm.at[pl.ds(base_b, _BPW)], xn_ab)
    pltpu.sync_copy(y_hbm.at[pl.ds(base_b, _BPW)], y_v)
    pltpu.sync_copy(idxn_hbm.at[pl.ds(wid * _BPW * _NCH, _BPW * _NCH)], idx_v)

    pltpu.async_copy(memab_hbm.at[y_v], vab_v, vsem).wait()
    pltpu.async_copy(meml_hbm.at[y_v], vl_v, vsem).wait()

    lanes = lax.broadcasted_iota(jnp.int32, (16,), 0)
    zero16 = jnp.zeros((16,), jnp.float32)
    col_of = lanes * 16

    def run_phase(mem_hbm, xn, v_v, dsq_hbm, pos_hbm):
        def pos_grp(g, carry):
            def pos_row(rr, tot):
                r = g * 16 + rr
                for j in range(_F // 16):
                    d = v_v[r, pl.ds(j * 16, 16)] - xn[r, pl.ds(j * 16, 16)]
                    acc = d * d if j == 0 else acc + d * d
                return jnp.where(lanes == rr, jnp.sum(acc), tot)
            tot = lax.fori_loop(0, 16, pos_row, zero16)
            pos_v[pl.ds(g * 16, 16)] = tot
            return carry
        lax.fori_loop(0, _BPW // 16, pos_grp, 0)
        pltpu.sync_copy(pos_v, pos_hbm.at[pl.ds(base_b, _BPW)])

        pltpu.async_copy(mem_hbm.at[idx_v.at[0]], buf0, sem0)

        def nb_body(b, carry):
            xr = [xn[b, pl.ds(j * 16, 16)] for j in range(_F // 16)]
            for c in range(_NCH):
                bufc = buf0 if c % 2 == 0 else buf1
                semc = sem0 if c % 2 == 0 else sem1
                if c + 1 < _NCH:
                    nbuf = buf1 if c % 2 == 0 else buf0
                    nsem = sem1 if c % 2 == 0 else sem0
                    pltpu.async_copy(
                        mem_hbm.at[idx_v.at[b * _NCH + c + 1]], nbuf, nsem)
                else:
                    @pl.when(b + 1 < _BPW)
                    def _next_b():
                        pltpu.async_copy(
                            mem_hbm.at[idx_v.at[(b + 1) * _NCH]], buf0, sem0)
                pltpu.make_async_copy(
                    mem_hbm.at[idx_v.at[b * _NCH + c]], bufc, semc).wait()

                def row_body(r, rc, _bufc=bufc):
                    for j in range(_F // 16):
                        d = _bufc[r, pl.ds(j * 16, 16)] - xr[j]
                        acc = d * d if j == 0 else acc + d * d
                    part[pl.ds(r * 16, 16)] = acc
                    return rc
                lax.fori_loop(0, _CH, row_body, 0)

                def grp_body(g, gc, _c=c):
                    base = g * 256 + col_of
                    tot = plsc.load_gather(part, [base])
                    for j in range(1, 16):
                        tot = tot + plsc.load_gather(part, [base + j])
                    dsq_v[pl.ds(b * _K + _c * _CH + g * 16, 16)] = tot
                    return gc
                lax.fori_loop(0, _CH // 16, grp_body, 0)
            return carry
        lax.fori_loop(0, _BPW, nb_body, 0)
        pltpu.sync_copy(dsq_v, dsq_hbm.at[pl.ds(base_b * _K, _BPW * _K)])

    run_phase(memab_hbm, xn_l, vab_v, dpab_hbm, ppab_hbm)
    run_phase(meml_hbm, xn_ab, vl_v, dpl_hbm, ppl_hbm)

    pltpu.sync_copy(vl_v, vl_hbm.at[pl.ds(base_b, _BPW)])
    pltpu.sync_copy(vab_v, vab_hbm.at[pl.ds(base_b, _BPW)])


def _sc_gather(l_n, ab_n, y, idx_neg, memory_l, memory_ab):
    mesh = plsc.VectorSubcoreMesh(core_axis_name="c", subcore_axis_name="s")
    f32 = jnp.float32
    out_type = [
        jax.ShapeDtypeStruct((_B * _K,), f32),
        jax.ShapeDtypeStruct((_B * _K,), f32),
        jax.ShapeDtypeStruct((_B,), f32),
        jax.ShapeDtypeStruct((_B,), f32),
        jax.ShapeDtypeStruct((_B, _F), f32),
        jax.ShapeDtypeStruct((_B, _F), f32),
    ]
    scratch = [
        pltpu.VMEM((_BPW, _F), f32),
        pltpu.VMEM((_BPW, _F), f32),
        pltpu.VMEM((_BPW,), jnp.int32),
        pltpu.VMEM((_BPW * _NCH, _CH), jnp.int32),
        pltpu.VMEM((_BPW, _F), f32),
        pltpu.VMEM((_BPW, _F), f32),
        pltpu.VMEM((_BPW * _K,), f32),
        pltpu.VMEM((_BPW,), f32),
        pltpu.VMEM((_CH * 16,), f32),
        pltpu.VMEM((_CH, _F), f32),
        pltpu.VMEM((_CH, _F), f32),
        pltpu.SemaphoreType.DMA,
        pltpu.SemaphoreType.DMA,
        pltpu.SemaphoreType.DMA,
    ]
    fn = pl.kernel(_sc_body, out_type=out_type, mesh=mesh,
                   scratch_types=scratch,
                   compiler_params=pltpu.CompilerParams(
                       needs_layout_passes=False,
                       skip_device_barrier=True),
                   cost_estimate=pl.CostEstimate(
                       flops=600_000_000,
                       transcendentals=0,
                       bytes_accessed=1_100_000_000))
    return fn(l_n, ab_n, y, idx_neg, memory_l, memory_ab)


def _finish_body(dsqab_ref, dsql_ref, posab_ref, posl_ref,
                 outl_ref, outab_ref):
    one = jnp.ones((_BB, 1), jnp.float32)

    def finish(dsq_ref, pos_ref, out_ref):
        pos = jnp.sqrt(pos_ref[...]).reshape(_BB, 1)
        d = jnp.sqrt(dsq_ref[...])
        o = jnp.maximum(1.0 + pos - d, 0.0)
        out_ref[...] = jnp.concatenate([one, o], axis=1)

    finish(dsqab_ref, posab_ref, outl_ref)
    finish(dsql_ref, posl_ref, outab_ref)


def _tc_finish(dsqab, dsql, posab, posl):
    f32 = jnp.float32
    dspec = pl.BlockSpec((_BB, _K), lambda g: (g, 0))
    pspec = pl.BlockSpec((_BB,), lambda g: (g,))
    ospec = pl.BlockSpec((_BB, _KP1), lambda g: (g, 0))
    return pl.pallas_call(
        _finish_body,
        grid=(_GB,),
        in_specs=[dspec, dspec, pspec, pspec],
        out_specs=[ospec, ospec],
        out_shape=[
            jax.ShapeDtypeStruct((_B, _KP1), f32),
            jax.ShapeDtypeStruct((_B, _KP1), f32),
        ],
    )(dsqab, dsql, posab, posl)


def _scatter_body(vl_ref, vab_ref, ln_ref, abn_ref, y_ref,
                  meml_ref, memab_ref, nml_ref, nmab_ref,
                  updl_scr, updab_scr, sem):
    wl = vl_ref[...] * _MOM + ln_ref[...] * (1.0 - _MOM)
    updl_scr[...] = wl / jnp.sqrt(jnp.sum(wl * wl, axis=1, keepdims=True))
    wab = vab_ref[...] * _MOM + abn_ref[...] * (1.0 - _MOM)
    updab_scr[...] = wab / jnp.sqrt(jnp.sum(wab * wab, axis=1, keepdims=True))

    def issue(i, carry):
        yi = y_ref[i]
        pltpu.make_async_copy(updl_scr.at[i], nml_ref.at[yi], sem).start()
        pltpu.make_async_copy(updab_scr.at[i], nmab_ref.at[yi], sem).start()
        return carry
    lax.fori_loop(0, _B, issue, 0)

    def drain(i, carry):
        yi = y_ref[i]
        pltpu.make_async_copy(updl_scr.at[i], nml_ref.at[yi], sem).wait()
        pltpu.make_async_copy(updab_scr.at[i], nmab_ref.at[yi], sem).wait()
        return carry
    lax.fori_loop(0, _B, drain, 0)


def _tc_scatter(vl, vab, l_n, ab_n, y, memory_l, memory_ab):
    f32 = jnp.float32
    vspec = pl.BlockSpec(memory_space=pltpu.VMEM)
    aspec = pl.BlockSpec(memory_space=pl.ANY)
    sspec = pl.BlockSpec(memory_space=pltpu.SMEM)
    return pl.pallas_call(
        _scatter_body,
        in_specs=[vspec, vspec, vspec, vspec, sspec, aspec, aspec],
        out_specs=[aspec, aspec],
        out_shape=[
            jax.ShapeDtypeStruct((_N, _F), f32),
            jax.ShapeDtypeStruct((_N, _F), f32),
        ],
        scratch_shapes=[
            pltpu.VMEM((_B, _F), f32),
            pltpu.VMEM((_B, _F), f32),
            pltpu.SemaphoreType.DMA,
        ],
        input_output_aliases={5: 0, 6: 1},
    )(vl, vab, l_n, ab_n, y, memory_l, memory_ab)


def kernel(l, ab, y, idx, memory_l, memory_ab):
    l_n, ab_n = pl.pallas_call(
        _norm_body,
        out_shape=[jax.ShapeDtypeStruct((_B, _F), jnp.float32)] * 2,
    )(l, ab)

    idx_neg = idx[:, 1:].reshape(_B * _NCH, _CH)

    dsqab, dsql, posab, posl, vl, vab = _sc_gather(
        l_n, ab_n, y, idx_neg, memory_l, memory_ab)

    out_l, out_ab = _tc_finish(
        dsqab.reshape(_B, _K), dsql.reshape(_B, _K), posab, posl)

    nml, nmab = _tc_scatter(vl, vab, l_n, ab_n, y, memory_l, memory_ab)

    return (out_l[..., None], out_ab[..., None], nml, nmab)


# speedup vs baseline: 17.5777x; 1.0476x over previous
import jax
import jax.numpy as jnp
from jax import lax
from jax.experimental import pallas as pl
from jax.experimental.pallas import tpu as pltpu
from jax.experimental.pallas import tpu_sc as plsc

_MOM = 0.5
_EPS = 1e-07

_B = 1024
_F = 128
_KP1 = 1025
_N = 1000000

_NC = 2
_NS = 16
_NW = _NC * _NS
_BPW = _B // _NW
_CH = 128
_K = _KP1 - 1
_NCH = _K // _CH
_GB = 8
_BB = _B // _GB
_RPW = (_N // _NW) // 8 * 8
_RTAIL = _N - _RPW * _NW


def _norm_body(l_ref, ab_ref, ln_ref, abn_ref):
    l = l_ref[...]
    ab = ab_ref[...]
    ln_ref[...] = l / (jnp.sqrt(jnp.sum(l * l, axis=1, keepdims=True)) + _EPS)
    abn_ref[...] = ab / (jnp.sqrt(jnp.sum(ab * ab, axis=1, keepdims=True)) + _EPS)


def _sc_body(ln_hbm, abn_hbm, y_hbm, idxn_hbm, meml_hbm, memab_hbm,
             dpab_hbm, dpl_hbm, ppab_hbm, ppl_hbm, vl_hbm, vab_hbm,
             xn_l, xn_ab, y_v, idx_v, vl_v, vab_v, dsq_v, pos_v, part,
             buf0, buf1, sem0, sem1, vsem):
    cid = lax.axis_index("c")
    sid = lax.axis_index("s")
    wid = sid * _NC + cid
    base_b = wid * _BPW

    pltpu.sync_copy(ln_hbm.at[pl.ds(base_b, _BPW)], xn_l)
    pltpu.sync_copy(abn_hbm.at[pl.ds(base_b, _BPW)], xn_ab)
    pltpu.sync_copy(y_hbm.at[pl.ds(base_b, _BPW)], y_v)
    pltpu.sync_copy(idxn_hbm.at[pl.ds(wid * _BPW * _NCH, _BPW * _NCH)], idx_v)

    pltpu.async_copy(memab_hbm.at[y_v], vab_v, vsem).wait()
    pltpu.async_copy(meml_hbm.at[y_v], vl_v, vsem).wait()

    lanes = lax.broadcasted_iota(jnp.int32, (16,), 0)
    zero16 = jnp.zeros((16,), jnp.float32)
    col_of = lanes * 17

    def run_phase(mem_hbm, xn, v_v, dsq_hbm, pos_hbm):
        def pos_grp(g, carry):
            def pos_row(rr, tot):
                r = g * 16 + rr
                for j in range(_F // 16):
                    d = v_v[r, pl.ds(j * 16, 16)] - xn[r, pl.ds(j * 16, 16)]
                    acc = d * d if j == 0 else acc + d * d
                return jnp.where(lanes == rr, jnp.sum(acc), tot)
            tot = lax.fori_loop(0, 16, pos_row, zero16)
            pos_v[pl.ds(g * 16, 16)] = tot
            return carry
        lax.fori_loop(0, _BPW // 16, pos_grp, 0)
        pltpu.sync_copy(pos_v, pos_hbm.at[pl.ds(base_b, _BPW)])

        pltpu.async_copy(mem_hbm.at[idx_v.at[0]], buf0, sem0)

        def nb_body(b, carry):
            xr = [xn[b, pl.ds(j * 16, 16)] for j in range(_F // 16)]
            for c in range(_NCH):
                bufc = buf0 if c % 2 == 0 else buf1
                semc = sem0 if c % 2 == 0 else sem1
                if c + 1 < _NCH:
                    nbuf = buf1 if c % 2 == 0 else buf0
                    nsem = sem1 if c % 2 == 0 else sem0
                    pltpu.async_copy(
                        mem_hbm.at[idx_v.at[b * _NCH + c + 1]], nbuf, nsem)
                else:
                    @pl.when(b + 1 < _BPW)
                    def _next_b():
                        pltpu.async_copy(
                            mem_hbm.at[idx_v.at[(b + 1) * _NCH]], buf0, sem0)
                pltpu.make_async_copy(
                    mem_hbm.at[idx_v.at[b * _NCH + c]], bufc, semc).wait()

                def row_body(r, rc, _bufc=bufc):
                    for j in range(_F // 16):
                        d = _bufc[r, pl.ds(j * 16, 16)] - xr[j]
                        acc = d * d if j == 0 else acc + d * d
                    part[pl.ds(r * 17, 16)] = acc
                    return rc
                lax.fori_loop(0, _CH, row_body, 0)

                def grp_body(g, gc, _c=c):
                    base = g * 272 + col_of
                    tot = plsc.load_gather(part, [base])
                    for j in range(1, 16):
                        tot = tot + plsc.load_gather(part, [base + j])
                    dsq_v[pl.ds(b * _K + _c * _CH + g * 16, 16)] = tot
                    return gc
                lax.fori_loop(0, _CH // 16, grp_body, 0)
            return carry
        lax.fori_loop(0, _BPW, nb_body, 0)
        pltpu.sync_copy(dsq_v, dsq_hbm.at[pl.ds(base_b * _K, _BPW * _K)])

    run_phase(memab_hbm, xn_l, vab_v, dpab_hbm, ppab_hbm)
    run_phase(meml_hbm, xn_ab, vl_v, dpl_hbm, ppl_hbm)

    pltpu.sync_copy(vl_v, vl_hbm.at[pl.ds(base_b, _BPW)])
    pltpu.sync_copy(vab_v, vab_hbm.at[pl.ds(base_b, _BPW)])


def _sc_gather(l_n, ab_n, y, idx_neg, memory_l, memory_ab):
    mesh = plsc.VectorSubcoreMesh(core_axis_name="c", subcore_axis_name="s")
    f32 = jnp.float32
    out_type = [
        jax.ShapeDtypeStruct((_B * _K,), f32),
        jax.ShapeDtypeStruct((_B * _K,), f32),
        jax.ShapeDtypeStruct((_B,), f32),
        jax.ShapeDtypeStruct((_B,), f32),
        jax.ShapeDtypeStruct((_B, _F), f32),
        jax.ShapeDtypeStruct((_B, _F), f32),
    ]
    scratch = [
        pltpu.VMEM((_BPW, _F), f32),
        pltpu.VMEM((_BPW, _F), f32),
        pltpu.VMEM((_BPW,), jnp.int32),
        pltpu.VMEM((_BPW * _NCH, _CH), jnp.int32),
        pltpu.VMEM((_BPW, _F), f32),
        pltpu.VMEM((_BPW, _F), f32),
        pltpu.VMEM((_BPW * _K,), f32),
        pltpu.VMEM((_BPW,), f32),
        pltpu.VMEM((_CH * 17,), f32),
        pltpu.VMEM((_CH, _F), f32),
        pltpu.VMEM((_CH, _F), f32),
        pltpu.SemaphoreType.DMA,
        pltpu.SemaphoreType.DMA,
        pltpu.SemaphoreType.DMA,
    ]
    fn = pl.kernel(_sc_body, out_type=out_type, mesh=mesh,
                   scratch_types=scratch,
                   compiler_params=pltpu.CompilerParams(
                       needs_layout_passes=False,
                       skip_device_barrier=True),
                   cost_estimate=pl.CostEstimate(
                       flops=600_000_000,
                       transcendentals=0,
                       bytes_accessed=1_100_000_000))
    return fn(l_n, ab_n, y, idx_neg, memory_l, memory_ab)


def _finish_body(dsqab_ref, dsql_ref, posab_ref, posl_ref,
                 outl_ref, outab_ref):
    one = jnp.ones((_BB, 1), jnp.float32)

    def finish(dsq_ref, pos_ref, out_ref):
        pos = jnp.sqrt(pos_ref[...]).reshape(_BB, 1)
        d = jnp.sqrt(dsq_ref[...])
        o = jnp.maximum(1.0 + pos - d, 0.0)
        out_ref[...] = jnp.concatenate([one, o], axis=1)

    finish(dsqab_ref, posab_ref, outl_ref)
    finish(dsql_ref, posl_ref, outab_ref)


def _tc_finish(dsqab, dsql, posab, posl):
    f32 = jnp.float32
    dspec = pl.BlockSpec((_BB, _K), lambda g: (g, 0))
    pspec = pl.BlockSpec((_BB,), lambda g: (g,))
    ospec = pl.BlockSpec((_BB, _KP1), lambda g: (g, 0))
    return pl.pallas_call(
        _finish_body,
        grid=(_GB,),
        in_specs=[dspec, dspec, pspec, pspec],
        out_specs=[ospec, ospec],
        out_shape=[
            jax.ShapeDtypeStruct((_B, _KP1), f32),
            jax.ShapeDtypeStruct((_B, _KP1), f32),
        ],
    )(dsqab, dsql, posab, posl)


def _scatter_body(vl_ref, vab_ref, ln_ref, abn_ref, y_ref,
                  meml_ref, memab_ref, nml_ref, nmab_ref,
                  updl_scr, updab_scr, sem):
    wl = vl_ref[...] * _MOM + ln_ref[...] * (1.0 - _MOM)
    updl_scr[...] = wl / jnp.sqrt(jnp.sum(wl * wl, axis=1, keepdims=True))
    wab = vab_ref[...] * _MOM + abn_ref[...] * (1.0 - _MOM)
    updab_scr[...] = wab / jnp.sqrt(jnp.sum(wab * wab, axis=1, keepdims=True))

    def issue(i, carry):
        yi = y_ref[i]
        pltpu.make_async_copy(updl_scr.at[i], nml_ref.at[yi], sem).start()
        pltpu.make_async_copy(updab_scr.at[i], nmab_ref.at[yi], sem).start()
        return carry
    lax.fori_loop(0, _B, issue, 0)

    def drain(i, carry):
        yi = y_ref[i]
        pltpu.make_async_copy(updl_scr.at[i], nml_ref.at[yi], sem).wait()
        pltpu.make_async_copy(updab_scr.at[i], nmab_ref.at[yi], sem).wait()
        return carry
    lax.fori_loop(0, _B, drain, 0)


def _tc_scatter(vl, vab, l_n, ab_n, y, memory_l, memory_ab):
    f32 = jnp.float32
    vspec = pl.BlockSpec(memory_space=pltpu.VMEM)
    aspec = pl.BlockSpec(memory_space=pl.ANY)
    sspec = pl.BlockSpec(memory_space=pltpu.SMEM)
    return pl.pallas_call(
        _scatter_body,
        in_specs=[vspec, vspec, vspec, vspec, sspec, aspec, aspec],
        out_specs=[aspec, aspec],
        out_shape=[
            jax.ShapeDtypeStruct((_N, _F), f32),
            jax.ShapeDtypeStruct((_N, _F), f32),
        ],
        scratch_shapes=[
            pltpu.VMEM((_B, _F), f32),
            pltpu.VMEM((_B, _F), f32),
            pltpu.SemaphoreType.DMA,
        ],
        input_output_aliases={5: 0, 6: 1},
    )(vl, vab, l_n, ab_n, y, memory_l, memory_ab)


def kernel(l, ab, y, idx, memory_l, memory_ab):
    l_n, ab_n = pl.pallas_call(
        _norm_body,
        out_shape=[jax.ShapeDtypeStruct((_B, _F), jnp.float32)] * 2,
    )(l, ab)

    idx_neg = idx[:, 1:].reshape(_B * _NCH, _CH)

    dsqab, dsql, posab, posl, vl, vab = _sc_gather(
        l_n, ab_n, y, idx_neg, memory_l, memory_ab)

    out_l, out_ab = _tc_finish(
        dsqab.reshape(_B, _K), dsql.reshape(_B, _K), posab, posl)

    nml, nmab = _tc_scatter(vl, vab, l_n, ab_n, y, memory_l, memory_ab)

    return (out_l[..., None], out_ab[..., None], nml, nmab)


# speedup vs baseline: 24.0845x; 1.3702x over previous
import jax
import jax.numpy as jnp
from jax import lax
from jax.experimental import pallas as pl
from jax.experimental.pallas import tpu as pltpu
from jax.experimental.pallas import tpu_sc as plsc

_MOM = 0.5
_EPS = 1e-07

_B = 1024
_F = 128
_KP1 = 1025
_N = 1000000

_NC = 2
_NS = 16
_NW = _NC * _NS
_BPW = _B // _NW
_CH = 128
_K = _KP1 - 1
_NCH = _K // _CH
_GB = 8
_BB = _B // _GB
_RPW = (_N // _NW) // 8 * 8
_RTAIL = _N - _RPW * _NW


def _norm_body(l_ref, ab_ref, ln_ref, abn_ref):
    l = l_ref[...]
    ab = ab_ref[...]
    ln_ref[...] = l / (jnp.sqrt(jnp.sum(l * l, axis=1, keepdims=True)) + _EPS)
    abn_ref[...] = ab / (jnp.sqrt(jnp.sum(ab * ab, axis=1, keepdims=True)) + _EPS)


def _sc_body(ln_hbm, abn_hbm, y_hbm, idxn_hbm, meml_hbm, memab_hbm,
             dpab_hbm, dpl_hbm, ppab_hbm, ppl_hbm, vl_hbm, vab_hbm,
             xn_l, xn_ab, y_v, idx_v, vl_v, vab_v, dsq_v, pos_v, part,
             buf0, buf1, sem0, sem1, vsem):
    cid = lax.axis_index("c")
    sid = lax.axis_index("s")
    wid = sid * _NC + cid
    base_b = wid * _BPW

    pltpu.sync_copy(ln_hbm.at[pl.ds(base_b, _BPW)], xn_l)
    pltpu.sync_copy(abn_hbm.at[pl.ds(base_b, _BPW)], xn_ab)
    pltpu.sync_copy(y_hbm.at[pl.ds(base_b, _BPW)], y_v)
    pltpu.sync_copy(idxn_hbm.at[pl.ds(wid * _BPW * _NCH, _BPW * _NCH)], idx_v)

    pltpu.async_copy(memab_hbm.at[y_v], vab_v, vsem).wait()
    pltpu.async_copy(meml_hbm.at[y_v], vl_v, vsem).wait()

    lanes = lax.broadcasted_iota(jnp.int32, (16,), 0)
    zero16 = jnp.zeros((16,), jnp.float32)
    col_of = lanes * 17

    def run_phase(mem_hbm, xn, v_v, dsq_hbm, pos_hbm):
        def pos_grp(g, carry):
            def pos_row(rr, tot):
                r = g * 16 + rr
                for j in range(_F // 16):
                    d = v_v[r, pl.ds(j * 16, 16)] - xn[r, pl.ds(j * 16, 16)]
                    acc = d * d if j == 0 else acc + d * d
                return jnp.where(lanes == rr, jnp.sum(acc), tot)
            tot = lax.fori_loop(0, 16, pos_row, zero16)
            pos_v[pl.ds(g * 16, 16)] = tot
            return carry
        lax.fori_loop(0, _BPW // 16, pos_grp, 0)
        pltpu.sync_copy(pos_v, pos_hbm.at[pl.ds(base_b, _BPW)])

        pltpu.async_copy(mem_hbm.at[idx_v.at[0]], buf0, sem0)

        def nb_body(b, carry):
            xr = [xn[b, pl.ds(j * 16, 16)] for j in range(_F // 16)]
            for c in range(_NCH):
                bufc = buf0 if c % 2 == 0 else buf1
                semc = sem0 if c % 2 == 0 else sem1
                if c + 1 < _NCH:
                    nbuf = buf1 if c % 2 == 0 else buf0
                    nsem = sem1 if c % 2 == 0 else sem0
                    pltpu.async_copy(
                        mem_hbm.at[idx_v.at[b * _NCH + c + 1]], nbuf, nsem)
                else:
                    @pl.when(b + 1 < _BPW)
                    def _next_b():
                        pltpu.async_copy(
                            mem_hbm.at[idx_v.at[(b + 1) * _NCH]], buf0, sem0)
                pltpu.make_async_copy(
                    mem_hbm.at[idx_v.at[b * _NCH + c]], bufc, semc).wait()

                def grp_body(g, gc, _bufc=bufc, _c=c):
                    def row_body(rr, tot):
                        r = g * 16 + rr
                        for j in range(_F // 16):
                            d = _bufc[r, pl.ds(j * 16, 16)] - xr[j]
                            acc = d * d if j == 0 else acc + d * d
                        return jnp.where(lanes == rr, jnp.sum(acc), tot)
                    tot = lax.fori_loop(0, 16, row_body, zero16)
                    dsq_v[pl.ds(b * _K + _c * _CH + g * 16, 16)] = tot
                    return gc
                lax.fori_loop(0, _CH // 16, grp_body, 0)
            return carry
        lax.fori_loop(0, _BPW, nb_body, 0)
        pltpu.sync_copy(dsq_v, dsq_hbm.at[pl.ds(base_b * _K, _BPW * _K)])

    run_phase(memab_hbm, xn_l, vab_v, dpab_hbm, ppab_hbm)
    run_phase(meml_hbm, xn_ab, vl_v, dpl_hbm, ppl_hbm)

    pltpu.sync_copy(vl_v, vl_hbm.at[pl.ds(base_b, _BPW)])
    pltpu.sync_copy(vab_v, vab_hbm.at[pl.ds(base_b, _BPW)])


def _sc_gather(l_n, ab_n, y, idx_neg, memory_l, memory_ab):
    mesh = plsc.VectorSubcoreMesh(core_axis_name="c", subcore_axis_name="s")
    f32 = jnp.float32
    out_type = [
        jax.ShapeDtypeStruct((_B * _K,), f32),
        jax.ShapeDtypeStruct((_B * _K,), f32),
        jax.ShapeDtypeStruct((_B,), f32),
        jax.ShapeDtypeStruct((_B,), f32),
        jax.ShapeDtypeStruct((_B, _F), f32),
        jax.ShapeDtypeStruct((_B, _F), f32),
    ]
    scratch = [
        pltpu.VMEM((_BPW, _F), f32),
        pltpu.VMEM((_BPW, _F), f32),
        pltpu.VMEM((_BPW,), jnp.int32),
        pltpu.VMEM((_BPW * _NCH, _CH), jnp.int32),
        pltpu.VMEM((_BPW, _F), f32),
        pltpu.VMEM((_BPW, _F), f32),
        pltpu.VMEM((_BPW * _K,), f32),
        pltpu.VMEM((_BPW,), f32),
        pltpu.VMEM((_CH * 17,), f32),
        pltpu.VMEM((_CH, _F), f32),
        pltpu.VMEM((_CH, _F), f32),
        pltpu.SemaphoreType.DMA,
        pltpu.SemaphoreType.DMA,
        pltpu.SemaphoreType.DMA,
    ]
    fn = pl.kernel(_sc_body, out_type=out_type, mesh=mesh,
                   scratch_types=scratch,
                   compiler_params=pltpu.CompilerParams(
                       needs_layout_passes=False,
                       skip_device_barrier=True),
                   cost_estimate=pl.CostEstimate(
                       flops=600_000_000,
                       transcendentals=0,
                       bytes_accessed=1_100_000_000))
    return fn(l_n, ab_n, y, idx_neg, memory_l, memory_ab)


def _finish_body(dsqab_ref, dsql_ref, posab_ref, posl_ref,
                 outl_ref, outab_ref):
    one = jnp.ones((_BB, 1), jnp.float32)

    def finish(dsq_ref, pos_ref, out_ref):
        pos = jnp.sqrt(pos_ref[...]).reshape(_BB, 1)
        d = jnp.sqrt(dsq_ref[...])
        o = jnp.maximum(1.0 + pos - d, 0.0)
        out_ref[...] = jnp.concatenate([one, o], axis=1)

    finish(dsqab_ref, posab_ref, outl_ref)
    finish(dsql_ref, posl_ref, outab_ref)


def _tc_finish(dsqab, dsql, posab, posl):
    f32 = jnp.float32
    dspec = pl.BlockSpec((_BB, _K), lambda g: (g, 0))
    pspec = pl.BlockSpec((_BB,), lambda g: (g,))
    ospec = pl.BlockSpec((_BB, _KP1), lambda g: (g, 0))
    return pl.pallas_call(
        _finish_body,
        grid=(_GB,),
        in_specs=[dspec, dspec, pspec, pspec],
        out_specs=[ospec, ospec],
        out_shape=[
            jax.ShapeDtypeStruct((_B, _KP1), f32),
            jax.ShapeDtypeStruct((_B, _KP1), f32),
        ],
    )(dsqab, dsql, posab, posl)


def _scatter_body(vl_ref, vab_ref, ln_ref, abn_ref, y_ref,
                  meml_ref, memab_ref, nml_ref, nmab_ref,
                  updl_scr, updab_scr, sem):
    wl = vl_ref[...] * _MOM + ln_ref[...] * (1.0 - _MOM)
    updl_scr[...] = wl / jnp.sqrt(jnp.sum(wl * wl, axis=1, keepdims=True))
    wab = vab_ref[...] * _MOM + abn_ref[...] * (1.0 - _MOM)
    updab_scr[...] = wab / jnp.sqrt(jnp.sum(wab * wab, axis=1, keepdims=True))

    def issue(i, carry):
        yi = y_ref[i]
        pltpu.make_async_copy(updl_scr.at[i], nml_ref.at[yi], sem).start()
        pltpu.make_async_copy(updab_scr.at[i], nmab_ref.at[yi], sem).start()
        return carry
    lax.fori_loop(0, _B, issue, 0)

    def drain(i, carry):
        yi = y_ref[i]
        pltpu.make_async_copy(updl_scr.at[i], nml_ref.at[yi], sem).wait()
        pltpu.make_async_copy(updab_scr.at[i], nmab_ref.at[yi], sem).wait()
        return carry
    lax.fori_loop(0, _B, drain, 0)


def _tc_scatter(vl, vab, l_n, ab_n, y, memory_l, memory_ab):
    f32 = jnp.float32
    vspec = pl.BlockSpec(memory_space=pltpu.VMEM)
    aspec = pl.BlockSpec(memory_space=pl.ANY)
    sspec = pl.BlockSpec(memory_space=pltpu.SMEM)
    return pl.pallas_call(
        _scatter_body,
        in_specs=[vspec, vspec, vspec, vspec, sspec, aspec, aspec],
        out_specs=[aspec, aspec],
        out_shape=[
            jax.ShapeDtypeStruct((_N, _F), f32),
            jax.ShapeDtypeStruct((_N, _F), f32),
        ],
        scratch_shapes=[
            pltpu.VMEM((_B, _F), f32),
            pltpu.VMEM((_B, _F), f32),
            pltpu.SemaphoreType.DMA,
        ],
        input_output_aliases={5: 0, 6: 1},
    )(vl, vab, l_n, ab_n, y, memory_l, memory_ab)


def kernel(l, ab, y, idx, memory_l, memory_ab):
    l_n, ab_n = pl.pallas_call(
        _norm_body,
        out_shape=[jax.ShapeDtypeStruct((_B, _F), jnp.float32)] * 2,
    )(l, ab)

    idx_neg = idx[:, 1:].reshape(_B * _NCH, _CH)

    dsqab, dsql, posab, posl, vl, vab = _sc_gather(
        l_n, ab_n, y, idx_neg, memory_l, memory_ab)

    out_l, out_ab = _tc_finish(
        dsqab.reshape(_B, _K), dsql.reshape(_B, _K), posab, posl)

    nml, nmab = _tc_scatter(vl, vab, l_n, ab_n, y, memory_l, memory_ab)

    return (out_l[..., None], out_ab[..., None], nml, nmab)


# speedup vs baseline: 24.2042x; 1.0050x over previous
import jax
import jax.numpy as jnp
from jax import lax
from jax.experimental import pallas as pl
from jax.experimental.pallas import tpu as pltpu
from jax.experimental.pallas import tpu_sc as plsc

_MOM = 0.5
_EPS = 1e-07

_B = 1024
_F = 128
_KP1 = 1025
_N = 1000000

_NC = 2
_NS = 16
_NW = _NC * _NS
_BPW = _B // _NW
_CH = 128
_K = _KP1 - 1
_NCH = _K // _CH
_GB = 8
_BB = _B // _GB
_RPW = (_N // _NW) // 8 * 8
_RTAIL = _N - _RPW * _NW


def _norm_body(l_ref, ab_ref, ln_ref, abn_ref):
    l = l_ref[...]
    ab = ab_ref[...]
    ln_ref[...] = l / (jnp.sqrt(jnp.sum(l * l, axis=1, keepdims=True)) + _EPS)
    abn_ref[...] = ab / (jnp.sqrt(jnp.sum(ab * ab, axis=1, keepdims=True)) + _EPS)


def _sc_body(ln_hbm, abn_hbm, y_hbm, idxn_hbm, meml_hbm, memab_hbm,
             dpab_hbm, dpl_hbm, ppab_hbm, ppl_hbm, vl_hbm, vab_hbm,
             xn_l, xn_ab, y_v, idx_v, vl_v, vab_v, dsq_v, pos_v, part,
             buf0, buf1, sem0, sem1, vsem):
    cid = lax.axis_index("c")
    sid = lax.axis_index("s")
    wid = sid * _NC + cid
    base_b = wid * _BPW

    pltpu.sync_copy(ln_hbm.at[pl.ds(base_b, _BPW)], xn_l)
    pltpu.sync_copy(abn_hbm.at[pl.ds(base_b, _BPW)], xn_ab)
    pltpu.sync_copy(y_hbm.at[pl.ds(base_b, _BPW)], y_v)
    pltpu.sync_copy(idxn_hbm.at[pl.ds(wid * _BPW * _NCH, _BPW * _NCH)], idx_v)

    pltpu.async_copy(memab_hbm.at[y_v], vab_v, vsem).wait()
    pltpu.async_copy(meml_hbm.at[y_v], vl_v, vsem).wait()

    lanes = lax.broadcasted_iota(jnp.int32, (16,), 0)
    zero16 = jnp.zeros((16,), jnp.float32)
    col_of = lanes * 17

    def run_phase(mem_hbm, xn, v_v, dsq_hbm, pos_hbm):
        def pos_grp(g, carry):
            def pos_row(rr, tot):
                r = g * 16 + rr
                for j in range(_F // 16):
                    d = v_v[r, pl.ds(j * 16, 16)] - xn[r, pl.ds(j * 16, 16)]
                    acc = d * d if j == 0 else acc + d * d
                return jnp.where(lanes == rr, jnp.sum(acc), tot)
            tot = lax.fori_loop(0, 16, pos_row, zero16)
            pos_v[pl.ds(g * 16, 16)] = tot
            return carry
        lax.fori_loop(0, _BPW // 16, pos_grp, 0)
        pltpu.sync_copy(pos_v, pos_hbm.at[pl.ds(base_b, _BPW)])

        pltpu.async_copy(mem_hbm.at[idx_v.at[0]], buf0, sem0)

        def nb_body(b, carry):
            xr = [xn[b, pl.ds(j * 16, 16)] for j in range(_F // 16)]
            for c in range(_NCH):
                bufc = buf0 if c % 2 == 0 else buf1
                semc = sem0 if c % 2 == 0 else sem1
                if c + 1 < _NCH:
                    nbuf = buf1 if c % 2 == 0 else buf0
                    nsem = sem1 if c % 2 == 0 else sem0
                    pltpu.async_copy(
                        mem_hbm.at[idx_v.at[b * _NCH + c + 1]], nbuf, nsem)
                else:
                    @pl.when(b + 1 < _BPW)
                    def _next_b():
                        pltpu.async_copy(
                            mem_hbm.at[idx_v.at[(b + 1) * _NCH]], buf0, sem0)
                pltpu.make_async_copy(
                    mem_hbm.at[idx_v.at[b * _NCH + c]], bufc, semc).wait()

                @plsc.parallel_loop(0, _CH // 16, unroll=2)
                def _grp(g, _bufc=bufc, _c=c):
                    def row_body(rr, tot):
                        r = g * 16 + rr
                        for j in range(_F // 16):
                            d = _bufc[r, pl.ds(j * 16, 16)] - xr[j]
                            acc = d * d if j == 0 else acc + d * d
                        return jnp.where(lanes == rr, jnp.sum(acc), tot)
                    tot = lax.fori_loop(0, 16, row_body, zero16)
                    dsq_v[pl.ds(b * _K + _c * _CH + g * 16, 16)] = tot
            return carry
        lax.fori_loop(0, _BPW, nb_body, 0)
        pltpu.sync_copy(dsq_v, dsq_hbm.at[pl.ds(base_b * _K, _BPW * _K)])

    run_phase(memab_hbm, xn_l, vab_v, dpab_hbm, ppab_hbm)
    run_phase(meml_hbm, xn_ab, vl_v, dpl_hbm, ppl_hbm)

    pltpu.sync_copy(vl_v, vl_hbm.at[pl.ds(base_b, _BPW)])
    pltpu.sync_copy(vab_v, vab_hbm.at[pl.ds(base_b, _BPW)])


def _sc_gather(l_n, ab_n, y, idx_neg, memory_l, memory_ab):
    mesh = plsc.VectorSubcoreMesh(core_axis_name="c", subcore_axis_name="s")
    f32 = jnp.float32
    out_type = [
        jax.ShapeDtypeStruct((_B * _K,), f32),
        jax.ShapeDtypeStruct((_B * _K,), f32),
        jax.ShapeDtypeStruct((_B,), f32),
        jax.ShapeDtypeStruct((_B,), f32),
        jax.ShapeDtypeStruct((_B, _F), f32),
        jax.ShapeDtypeStruct((_B, _F), f32),
    ]
    scratch = [
        pltpu.VMEM((_BPW, _F), f32),
        pltpu.VMEM((_BPW, _F), f32),
        pltpu.VMEM((_BPW,), jnp.int32),
        pltpu.VMEM((_BPW * _NCH, _CH), jnp.int32),
        pltpu.VMEM((_BPW, _F), f32),
        pltpu.VMEM((_BPW, _F), f32),
        pltpu.VMEM((_BPW * _K,), f32),
        pltpu.VMEM((_BPW,), f32),
        pltpu.VMEM((_CH * 17,), f32),
        pltpu.VMEM((_CH, _F), f32),
        pltpu.VMEM((_CH, _F), f32),
        pltpu.SemaphoreType.DMA,
        pltpu.SemaphoreType.DMA,
        pltpu.SemaphoreType.DMA,
    ]
    fn = pl.kernel(_sc_body, out_type=out_type, mesh=mesh,
                   scratch_types=scratch,
                   compiler_params=pltpu.CompilerParams(
                       needs_layout_passes=False,
                       skip_device_barrier=True),
                   cost_estimate=pl.CostEstimate(
                       flops=600_000_000,
                       transcendentals=0,
                       bytes_accessed=1_100_000_000))
    return fn(l_n, ab_n, y, idx_neg, memory_l, memory_ab)


def _finish_body(dsqab_ref, dsql_ref, posab_ref, posl_ref,
                 outl_ref, outab_ref):
    one = jnp.ones((_BB, 1), jnp.float32)

    def finish(dsq_ref, pos_ref, out_ref):
        pos = jnp.sqrt(pos_ref[...]).reshape(_BB, 1)
        d = jnp.sqrt(dsq_ref[...])
        o = jnp.maximum(1.0 + pos - d, 0.0)
        out_ref[...] = jnp.concatenate([one, o], axis=1)

    finish(dsqab_ref, posab_ref, outl_ref)
    finish(dsql_ref, posl_ref, outab_ref)


def _tc_finish(dsqab, dsql, posab, posl):
    f32 = jnp.float32
    dspec = pl.BlockSpec((_BB, _K), lambda g: (g, 0))
    pspec = pl.BlockSpec((_BB,), lambda g: (g,))
    ospec = pl.BlockSpec((_BB, _KP1), lambda g: (g, 0))
    return pl.pallas_call(
        _finish_body,
        grid=(_GB,),
        in_specs=[dspec, dspec, pspec, pspec],
        out_specs=[ospec, ospec],
        out_shape=[
            jax.ShapeDtypeStruct((_B, _KP1), f32),
            jax.ShapeDtypeStruct((_B, _KP1), f32),
        ],
    )(dsqab, dsql, posab, posl)


def _scatter_body(vl_ref, vab_ref, ln_ref, abn_ref, y_ref,
                  meml_ref, memab_ref, nml_ref, nmab_ref,
                  updl_scr, updab_scr, sem):
    wl = vl_ref[...] * _MOM + ln_ref[...] * (1.0 - _MOM)
    updl_scr[...] = wl / jnp.sqrt(jnp.sum(wl * wl, axis=1, keepdims=True))
    wab = vab_ref[...] * _MOM + abn_ref[...] * (1.0 - _MOM)
    updab_scr[...] = wab / jnp.sqrt(jnp.sum(wab * wab, axis=1, keepdims=True))

    def issue(i, carry):
        yi = y_ref[i]
        pltpu.make_async_copy(updl_scr.at[i], nml_ref.at[yi], sem).start()
        pltpu.make_async_copy(updab_scr.at[i], nmab_ref.at[yi], sem).start()
        return carry
    lax.fori_loop(0, _B, issue, 0)

    def drain(i, carry):
        yi = y_ref[i]
        pltpu.make_async_copy(updl_scr.at[i], nml_ref.at[yi], sem).wait()
        pltpu.make_async_copy(updab_scr.at[i], nmab_ref.at[yi], sem).wait()
        return carry
    lax.fori_loop(0, _B, drain, 0)


def _tc_scatter(vl, vab, l_n, ab_n, y, memory_l, memory_ab):
    f32 = jnp.float32
    vspec = pl.BlockSpec(memory_space=pltpu.VMEM)
    aspec = pl.BlockSpec(memory_space=pl.ANY)
    sspec = pl.BlockSpec(memory_space=pltpu.SMEM)
    return pl.pallas_call(
        _scatter_body,
        in_specs=[vspec, vspec, vspec, vspec, sspec, aspec, aspec],
        out_specs=[aspec, aspec],
        out_shape=[
            jax.ShapeDtypeStruct((_N, _F), f32),
            jax.ShapeDtypeStruct((_N, _F), f32),
        ],
        scratch_shapes=[
            pltpu.VMEM((_B, _F), f32),
            pltpu.VMEM((_B, _F), f32),
            pltpu.SemaphoreType.DMA,
        ],
        input_output_aliases={5: 0, 6: 1},
    )(vl, vab, l_n, ab_n, y, memory_l, memory_ab)


def kernel(l, ab, y, idx, memory_l, memory_ab):
    l_n, ab_n = pl.pallas_call(
        _norm_body,
        out_shape=[jax.ShapeDtypeStruct((_B, _F), jnp.float32)] * 2,
    )(l, ab)

    idx_neg = idx[:, 1:].reshape(_B * _NCH, _CH)

    dsqab, dsql, posab, posl, vl, vab = _sc_gather(
        l_n, ab_n, y, idx_neg, memory_l, memory_ab)

    out_l, out_ab = _tc_finish(
        dsqab.reshape(_B, _K), dsql.reshape(_B, _K), posab, posl)

    nml, nmab = _tc_scatter(vl, vab, l_n, ab_n, y, memory_l, memory_ab)

    return (out_l[..., None], out_ab[..., None], nml, nmab)


# speedup vs baseline: 24.2501x; 1.0019x over previous
import jax
import jax.numpy as jnp
from jax import lax
from jax.experimental import pallas as pl
from jax.experimental.pallas import tpu as pltpu
from jax.experimental.pallas import tpu_sc as plsc

_MOM = 0.5
_EPS = 1e-07

_B = 1024
_F = 128
_KP1 = 1025
_N = 1000000

_NC = 2
_NS = 16
_NW = _NC * _NS
_BPW = _B // _NW
_CH = 128
_K = _KP1 - 1
_NCH = _K // _CH
_GB = 8
_BB = _B // _GB


def _norm_body(l_ref, ab_ref, ln_ref, abn_ref):
    l = l_ref[...]
    ab = ab_ref[...]
    ln_ref[...] = l / (jnp.sqrt(jnp.sum(l * l, axis=1, keepdims=True)) + _EPS)
    abn_ref[...] = ab / (jnp.sqrt(jnp.sum(ab * ab, axis=1, keepdims=True)) + _EPS)


def _sc_body(ln_hbm, abn_hbm, y_hbm, idxn_hbm, meml_hbm, memab_hbm,
             dpab_hbm, dpl_hbm, ppab_hbm, ppl_hbm, vl_hbm, vab_hbm,
             xn_l, xn_ab, y_v, idx_v, vl_v, vab_v, dsq_v, pos_v,
             buf0, buf1, sem0, sem1, vsem):
    cid = lax.axis_index("c")
    sid = lax.axis_index("s")
    wid = sid * _NC + cid
    base_b = wid * _BPW

    pltpu.sync_copy(ln_hbm.at[pl.ds(base_b, _BPW)], xn_l)
    pltpu.sync_copy(abn_hbm.at[pl.ds(base_b, _BPW)], xn_ab)
    pltpu.sync_copy(y_hbm.at[pl.ds(base_b, _BPW)], y_v)
    pltpu.sync_copy(idxn_hbm.at[pl.ds(wid * _BPW * _NCH, _BPW * _NCH)], idx_v)

    pltpu.async_copy(memab_hbm.at[y_v], vab_v, vsem).wait()
    pltpu.async_copy(meml_hbm.at[y_v], vl_v, vsem).wait()

    lanes = lax.broadcasted_iota(jnp.int32, (16,), 0)
    zero16 = jnp.zeros((16,), jnp.float32)

    def run_phase(mem_hbm, xn, v_v, dsq_hbm, pos_hbm):
        def pos_grp(g, carry):
            def pos_row(rr, tot):
                r = g * 16 + rr
                for j in range(_F // 16):
                    d = v_v[r, pl.ds(j * 16, 16)] - xn[r, pl.ds(j * 16, 16)]
                    acc = d * d if j == 0 else acc + d * d
                return jnp.where(lanes == rr, jnp.sum(acc), tot)
            tot = lax.fori_loop(0, 16, pos_row, zero16)
            pos_v[pl.ds(g * 16, 16)] = tot
            return carry
        lax.fori_loop(0, _BPW // 16, pos_grp, 0)
        pltpu.sync_copy(pos_v, pos_hbm.at[pl.ds(base_b, _BPW)])

        pltpu.async_copy(mem_hbm.at[idx_v.at[0]], buf0, sem0)

        def nb_body(b, carry):
            xr = [xn[b, pl.ds(j * 16, 16)] for j in range(_F // 16)]
            for c in range(_NCH):
                bufc = buf0 if c % 2 == 0 else buf1
                semc = sem0 if c % 2 == 0 else sem1
                if c + 1 < _NCH:
                    nbuf = buf1 if c % 2 == 0 else buf0
                    nsem = sem1 if c % 2 == 0 else sem0
                    pltpu.async_copy(
                        mem_hbm.at[idx_v.at[b * _NCH + c + 1]], nbuf, nsem)
                else:
                    @pl.when(b + 1 < _BPW)
                    def _next_b():
                        pltpu.async_copy(
                            mem_hbm.at[idx_v.at[(b + 1) * _NCH]], buf0, sem0)
                pltpu.make_async_copy(
                    mem_hbm.at[idx_v.at[b * _NCH + c]], bufc, semc).wait()

                @plsc.parallel_loop(0, _CH // 16, unroll=2)
                def _grp(g, _bufc=bufc, _c=c):
                    def row_body(rr, tot):
                        r = g * 16 + rr
                        for j in range(_F // 16):
                            d = _bufc[r, pl.ds(j * 16, 16)] - xr[j]
                            acc = d * d if j == 0 else acc + d * d
                        return jnp.where(lanes == rr, jnp.sum(acc), tot)
                    tot = lax.fori_loop(0, 16, row_body, zero16)
                    dsq_v[pl.ds(b * _K + _c * _CH + g * 16, 16)] = tot
            return carry
        lax.fori_loop(0, _BPW, nb_body, 0)
        pltpu.sync_copy(dsq_v, dsq_hbm.at[pl.ds(base_b * _K, _BPW * _K)])

    run_phase(memab_hbm, xn_l, vab_v, dpab_hbm, ppab_hbm)
    run_phase(meml_hbm, xn_ab, vl_v, dpl_hbm, ppl_hbm)

    pltpu.sync_copy(vl_v, vl_hbm.at[pl.ds(base_b, _BPW)])
    pltpu.sync_copy(vab_v, vab_hbm.at[pl.ds(base_b, _BPW)])


def _sc_gather(l_n, ab_n, y, idx_neg, memory_l, memory_ab):
    mesh = plsc.VectorSubcoreMesh(core_axis_name="c", subcore_axis_name="s")
    f32 = jnp.float32
    out_type = [
        jax.ShapeDtypeStruct((_B * _K,), f32),
        jax.ShapeDtypeStruct((_B * _K,), f32),
        jax.ShapeDtypeStruct((_B,), f32),
        jax.ShapeDtypeStruct((_B,), f32),
        jax.ShapeDtypeStruct((_B, _F), f32),
        jax.ShapeDtypeStruct((_B, _F), f32),
    ]
    scratch = [
        pltpu.VMEM((_BPW, _F), f32),
        pltpu.VMEM((_BPW, _F), f32),
        pltpu.VMEM((_BPW,), jnp.int32),
        pltpu.VMEM((_BPW * _NCH, _CH), jnp.int32),
        pltpu.VMEM((_BPW, _F), f32),
        pltpu.VMEM((_BPW, _F), f32),
        pltpu.VMEM((_BPW * _K,), f32),
        pltpu.VMEM((_BPW,), f32),
        pltpu.VMEM((_CH, _F), f32),
        pltpu.VMEM((_CH, _F), f32),
        pltpu.SemaphoreType.DMA,
        pltpu.SemaphoreType.DMA,
        pltpu.SemaphoreType.DMA,
    ]
    fn = pl.kernel(_sc_body, out_type=out_type, mesh=mesh,
                   scratch_types=scratch,
                   compiler_params=pltpu.CompilerParams(
                       needs_layout_passes=False,
                       skip_device_barrier=True),
                   cost_estimate=pl.CostEstimate(
                       flops=600_000_000,
                       transcendentals=0,
                       bytes_accessed=1_100_000_000))
    return fn(l_n, ab_n, y, idx_neg, memory_l, memory_ab)


def _finish_body(dsqab_ref, dsql_ref, posab_ref, posl_ref,
                 outl_ref, outab_ref):
    one = jnp.ones((_BB, 1), jnp.float32)

    def finish(dsq_ref, pos_ref, out_ref):
        pos = jnp.sqrt(pos_ref[...]).reshape(_BB, 1)
        d = jnp.sqrt(dsq_ref[...])
        o = jnp.maximum(1.0 + pos - d, 0.0)
        out_ref[...] = jnp.concatenate([one, o], axis=1)

    finish(dsqab_ref, posab_ref, outl_ref)
    finish(dsql_ref, posl_ref, outab_ref)


def _tc_finish(dsqab, dsql, posab, posl):
    f32 = jnp.float32
    dspec = pl.BlockSpec((_BB, _K), lambda g: (g, 0))
    pspec = pl.BlockSpec((_BB,), lambda g: (g,))
    ospec = pl.BlockSpec((_BB, _KP1), lambda g: (g, 0))
    return pl.pallas_call(
        _finish_body,
        grid=(_GB,),
        in_specs=[dspec, dspec, pspec, pspec],
        out_specs=[ospec, ospec],
        out_shape=[
            jax.ShapeDtypeStruct((_B, _KP1), f32),
            jax.ShapeDtypeStruct((_B, _KP1), f32),
        ],
    )(dsqab, dsql, posab, posl)


def _scatter_body(vl_ref, vab_ref, ln_ref, abn_ref, y_ref,
                  meml_ref, memab_ref, nml_ref, nmab_ref,
                  updl_scr, updab_scr, sem):
    wl = vl_ref[...] * _MOM + ln_ref[...] * (1.0 - _MOM)
    updl_scr[...] = wl / jnp.sqrt(jnp.sum(wl * wl, axis=1, keepdims=True))
    wab = vab_ref[...] * _MOM + abn_ref[...] * (1.0 - _MOM)
    updab_scr[...] = wab / jnp.sqrt(jnp.sum(wab * wab, axis=1, keepdims=True))

    def issue(i, carry):
        yi = y_ref[i]
        pltpu.make_async_copy(updl_scr.at[i], nml_ref.at[yi], sem).start()
        pltpu.make_async_copy(updab_scr.at[i], nmab_ref.at[yi], sem).start()
        return carry
    lax.fori_loop(0, _B, issue, 0)

    def drain(i, carry):
        yi = y_ref[i]
        pltpu.make_async_copy(updl_scr.at[i], nml_ref.at[yi], sem).wait()
        pltpu.make_async_copy(updab_scr.at[i], nmab_ref.at[yi], sem).wait()
        return carry
    lax.fori_loop(0, _B, drain, 0)


def _tc_scatter(vl, vab, l_n, ab_n, y, memory_l, memory_ab):
    f32 = jnp.float32
    vspec = pl.BlockSpec(memory_space=pltpu.VMEM)
    aspec = pl.BlockSpec(memory_space=pl.ANY)
    sspec = pl.BlockSpec(memory_space=pltpu.SMEM)
    return pl.pallas_call(
        _scatter_body,
        in_specs=[vspec, vspec, vspec, vspec, sspec, aspec, aspec],
        out_specs=[aspec, aspec],
        out_shape=[
            jax.ShapeDtypeStruct((_N, _F), f32),
            jax.ShapeDtypeStruct((_N, _F), f32),
        ],
        scratch_shapes=[
            pltpu.VMEM((_B, _F), f32),
            pltpu.VMEM((_B, _F), f32),
            pltpu.SemaphoreType.DMA,
        ],
        input_output_aliases={5: 0, 6: 1},
    )(vl, vab, l_n, ab_n, y, memory_l, memory_ab)


def kernel(l, ab, y, idx, memory_l, memory_ab):
    l_n, ab_n = pl.pallas_call(
        _norm_body,
        out_shape=[jax.ShapeDtypeStruct((_B, _F), jnp.float32)] * 2,
    )(l, ab)

    idx_neg = idx[:, 1:].reshape(_B * _NCH, _CH)

    dsqab, dsql, posab, posl, vl, vab = _sc_gather(
        l_n, ab_n, y, idx_neg, memory_l, memory_ab)

    out_l, out_ab = _tc_finish(
        dsqab.reshape(_B, _K), dsql.reshape(_B, _K), posab, posl)

    nml, nmab = _tc_scatter(vl, vab, l_n, ab_n, y, memory_l, memory_ab)

    return (out_l[..., None], out_ab[..., None], nml, nmab)
